# Initial kernel scaffold; baseline (speedup 1.0000x reference)
import jax
import jax.numpy as jnp
from jax.experimental import pallas as pl


def kernel(x, edge_index, batch, Wg0, bg0, Wg1, bg1, Wg2, bg2, Wg3, bg3, Wg4, bg4, Wh0, bh0, Wh1, bh1, Wh2, bh2):
    raise NotImplementedError("write your pallas kernel here")


# speedup vs baseline: 16.8831x; 16.8831x over previous
import functools

import jax
import jax.numpy as jnp
from jax import lax
from jax.experimental import pallas as pl
from jax.experimental.pallas import tpu as pltpu
from jax.experimental.pallas import tpu_sc as plsc

N = 10000
E = 320000
H = 128
NB = 16
NCL = 5
NC80 = NB * NCL
KM_ITERS = 10

NPAD = 10240
NWORK = 32
EPW = E // NWORK
CHUNK = 80
ROUNDS = EPW // CHUNK

_HIGH = jax.lax.Precision.HIGHEST


def _dot(a, b, dims):
    return jax.lax.dot_general(a, b, (dims, ((), ())), precision=_HIGH,
                               preferred_element_type=jnp.float32)


@functools.cache
def _sc_deg_kernel():
    mesh = plsc.VectorSubcoreMesh(core_axis_name="c", subcore_axis_name="s")
    return functools.partial(
        pl.kernel, mesh=mesh,
        out_type=jax.ShapeDtypeStruct((2, NPAD, H), jnp.float32),
        scratch_types=[
            pltpu.VMEM((ROUNDS, CHUNK), jnp.int32),
            pltpu.VMEM((CHUNK, H), jnp.float32),
            pltpu.VMEM_SHARED((NPAD, H), jnp.float32),
        ],
    )(_sc_deg_body)


def _sc_deg_body(dst3, ones_h, zeros_h, out, dstv, onesv, acc):
    c = lax.axis_index("c")
    s = lax.axis_index("s")
    w = c * 16 + s
    rows = NPAD // 16
    pltpu.sync_copy(dst3.at[w], dstv)
    pltpu.sync_copy(ones_h, onesv)
    pltpu.sync_copy(zeros_h.at[pl.ds(s * rows, rows)], acc.at[pl.ds(s * rows, rows)])
    plsc.subcore_barrier()

    def body(j, carry):
        pltpu.sync_copy(onesv, acc.at[dstv.at[j]], add=True)
        return carry

    lax.fori_loop(0, ROUNDS, body, 0)
    plsc.subcore_barrier()
    pltpu.sync_copy(acc.at[pl.ds(s * rows, rows)], out.at[c, pl.ds(s * rows, rows)])


@functools.cache
def _sc_agg_kernel():
    mesh = plsc.VectorSubcoreMesh(core_axis_name="c", subcore_axis_name="s")
    return functools.partial(
        pl.kernel, mesh=mesh,
        out_type=jax.ShapeDtypeStruct((2, NPAD, H), jnp.float32),
        scratch_types=[
            pltpu.VMEM((ROUNDS, CHUNK), jnp.int32),
            pltpu.VMEM((ROUNDS, CHUNK), jnp.int32),
            pltpu.VMEM((CHUNK, H), jnp.float32),
            pltpu.VMEM_SHARED((NPAD, H), jnp.float32),
            pltpu.SemaphoreType.DMA,
        ],
    )(_sc_agg_body)


def _sc_agg_body(y, src3, dst3, zeros_h, out, srcv, dstv, rowsv, acc, sem):
    c = lax.axis_index("c")
    s = lax.axis_index("s")
    w = c * 16 + s
    rows = NPAD // 16
    pltpu.sync_copy(src3.at[w], srcv)
    pltpu.sync_copy(dst3.at[w], dstv)
    pltpu.sync_copy(zeros_h.at[pl.ds(s * rows, rows)], acc.at[pl.ds(s * rows, rows)])
    plsc.subcore_barrier()

    def body(j, carry):
        pltpu.async_copy(y.at[srcv.at[j]], rowsv, sem).wait()
        pltpu.sync_copy(rowsv, acc.at[dstv.at[j]], add=True)
        return carry

    lax.fori_loop(0, ROUNDS, body, 0)
    plsc.subcore_barrier()
    pltpu.sync_copy(acc.at[pl.ds(s * rows, rows)], out.at[c, pl.ds(s * rows, rows)])


@functools.cache
def _sc_count_kernel():
    mesh = plsc.VectorSubcoreMesh(core_axis_name="c", subcore_axis_name="s")
    return functools.partial(
        pl.kernel, mesh=mesh,
        compiler_params=pltpu.CompilerParams(needs_layout_passes=False),
        out_type=jax.ShapeDtypeStruct((2, NC80 * NC80, H), jnp.float32),
        scratch_types=[
            pltpu.VMEM((N,), jnp.int32),
            pltpu.VMEM((EPW,), jnp.int32),
            pltpu.VMEM((EPW,), jnp.int32),
            pltpu.VMEM((1, CHUNK), jnp.int32),
            pltpu.VMEM((CHUNK, H), jnp.float32),
            pltpu.VMEM_SHARED((NC80 * NC80, H), jnp.float32),
        ],
    )(_sc_count_body)


def _sc_count_body(seg, src4, dst4, ones_h, zeros_h, out, segv, srcv, dstv, idxv,
                   onesv, acc):
    c = lax.axis_index("c")
    s = lax.axis_index("s")
    w = c * 16 + s
    rows = NC80 * NC80 // 16
    pltpu.sync_copy(seg, segv)
    pltpu.sync_copy(src4.at[w], srcv)
    pltpu.sync_copy(dst4.at[w], dstv)
    pltpu.sync_copy(ones_h, onesv)
    pltpu.sync_copy(zeros_h.at[pl.ds(s * rows, rows)], acc.at[pl.ds(s * rows, rows)])
    plsc.subcore_barrier()

    groups = CHUNK // 16

    def body(j, carry):
        for g in range(groups):
            base = (j * groups + g) * 16
            s16 = srcv[pl.ds(base, 16)]
            d16 = dstv[pl.ds(base, 16)]
            cs = plsc.load_gather(segv, [s16])
            cd = plsc.load_gather(segv, [d16])
            idxv.at[0][pl.ds(g * 16, 16)] = cs * NC80 + cd
        pltpu.sync_copy(onesv, acc.at[idxv.at[0]], add=True)
        return carry

    lax.fori_loop(0, ROUNDS, body, 0)
    plsc.subcore_barrier()
    pltpu.sync_copy(acc.at[pl.ds(s * rows, rows)], out.at[c, pl.ds(s * rows, rows)])


def _rs_from_deg(deg0, deg1):
    deg = deg0[:, 0:1] + deg1[:, 0:1] + 1.0
    rs = jax.lax.rsqrt(deg)
    return rs, rs * rs


def _tc_y_body(x, w0, b0, deg0, deg1, y_out, sb_out):
    rs, invd = _rs_from_deg(deg0[...], deg1[...])
    xw = _dot(x[...], w0[...], ((1,), (0,)))
    y_out[...] = xw * rs
    sb_out[...] = xw * invd + b0[...][None, :]


def _tc_hrelu_body(acca, accb, sb, deg0, deg1, h_out):
    rs, _ = _rs_from_deg(deg0[...], deg1[...])
    h_out[...] = jax.nn.relu(rs * (acca[...] + accb[...]) + sb[...])


_KB = 2000
_KGRID = N // _KB


def _tc_kmeans_body(h_ref, bat_ref, seg_out, cent_out, sums_scr, cnts_scr,
                    cent_scr):
    t = pl.program_id(0)
    b = pl.program_id(1)
    h = h_ref[...]
    bat = bat_ref[...]
    iota_g = jax.lax.broadcasted_iota(jnp.int32, (_KB, NB), 1)
    in_graph = bat == iota_g
    oh_g = jnp.where(in_graph, 1.0, 0.0)
    ones_col = jnp.ones((_KB, 1), jnp.float32)
    rowid = jax.lax.broadcasted_iota(jnp.int32, (_KB, 1), 0) + b * _KB
    init_lab = rowid % NCL

    dmin = None
    lab = jnp.zeros((_KB, 1), jnp.int32)
    for j in range(NCL):
        cj = _dot(oh_g, cent_scr[j], ((1,), (0,)))
        diff = h - cj
        dj = jnp.sum(diff * diff, axis=1, keepdims=True)
        if dmin is None:
            dmin = dj
        else:
            take = dj < dmin
            lab = jnp.where(take, j, lab)
            dmin = jnp.where(take, dj, dmin)
    label = jnp.where(t == 0, init_lab, lab)
    seg_out[...] = bat * NCL + label

    for j in range(NCL):
        aj = jnp.where(in_graph & (label == j), 1.0, 0.0)
        part = _dot(aj, h, ((0,), (0,)))
        pc = _dot(aj, ones_col, ((0,), (0,)))

        @pl.when(b == 0)
        def _():
            sums_scr[j] = part
            cnts_scr[j] = pc

        @pl.when(b > 0)
        def _():
            sums_scr[j] = sums_scr[j] + part
            cnts_scr[j] = cnts_scr[j] + pc

    @pl.when(b == _KGRID - 1)
    def _():
        for j in range(NCL):
            cent_scr[j] = sums_scr[j] / jnp.maximum(cnts_scr[j], 1.0)

    @pl.when((t == KM_ITERS) & (b == _KGRID - 1))
    def _():
        for j in range(NCL):
            cent_out[j] = cent_scr[j]


def _tc_kmeans(h2, batch2):
    st = jax.ShapeDtypeStruct
    return pl.pallas_call(
        _tc_kmeans_body,
        grid=(KM_ITERS + 1, _KGRID),
        in_specs=[pl.BlockSpec((_KB, H), lambda t, b: (b, 0)),
                  pl.BlockSpec((_KB, 1), lambda t, b: (b, 0))],
        out_specs=[pl.BlockSpec((_KB, 1), lambda t, b: (b, 0)),
                   pl.BlockSpec((NCL, NB, H), lambda t, b: (0, 0, 0))],
        out_shape=[st((N, 1), jnp.int32), st((NCL, NB, H), jnp.float32)],
        scratch_shapes=[pltpu.VMEM((NCL, NB, H), jnp.float32),
                        pltpu.VMEM((NCL, NB, 1), jnp.float32),
                        pltpu.VMEM((NCL, NB, H), jnp.float32)],
    )(h2, batch2)


def _tc_coarse_body(c0, c1, hc, w2, b2, w3, b3, w4, b4, wh0, bh0, wh1, bh1,
                    wh2, bh2, out):
    craw = c0[...] + c1[...]
    r = jax.lax.broadcasted_iota(jnp.int32, (NC80, NC80), 0)
    col = jax.lax.broadcasted_iota(jnp.int32, (NC80, NC80), 1)
    ct = jnp.where(r == col, 0.0, craw)
    ones_col = jnp.ones((NC80, 1), jnp.float32)
    degc = _dot(ct, ones_col, ((0,), (0,))) + 1.0
    degr = jnp.sum(ct, axis=0, keepdims=True) + 1.0
    rsc_c = jax.lax.rsqrt(degc)
    rsc_r = jax.lax.rsqrt(degr)
    invdc = rsc_c * rsc_c
    m = ct * rsc_c * rsc_r

    def layer(hm, w, b):
        xw = _dot(hm, w[...], ((1,), (0,)))
        agg = _dot(m, xw, ((0,), (0,)))
        return jax.nn.relu(agg + xw * invdc + b[...][None, :])

    h = layer(hc[...], w2, b2)
    h = layer(h, w3, b3)
    h = layer(h, w4, b4)

    pr = jax.lax.broadcasted_iota(jnp.int32, (NB, NC80), 0)
    pc = jax.lax.broadcasted_iota(jnp.int32, (NB, NC80), 1)
    pool = jnp.where(pc // NCL == pr, 1.0 / NCL, 0.0)
    g = _dot(pool, h, ((1,), (0,)))
    g = jax.nn.gelu(_dot(g, wh0[...], ((1,), (0,))) + bh0[...][None, :])
    g = jax.nn.gelu(_dot(g, wh1[...], ((1,), (0,))) + bh1[...][None, :])
    out[...] = _dot(g, wh2[...], ((1,), (0,))) + bh2[...][None, :]


def _tc_call(body, out_shapes, *args):
    return pl.pallas_call(body, out_shape=out_shapes)(*args)


_RB = 2000
_GRID = N // _RB


def _row_spec():
    return pl.BlockSpec((_RB, H), lambda i: (i, 0))


def _deg_spec():
    return pl.BlockSpec((_RB, H), lambda i: (i, 0))


def _full_spec(shape):
    return pl.BlockSpec(shape, lambda i: tuple(0 for _ in shape))


def _tc_y(xin, w, b, deg0, deg1):
    st = jax.ShapeDtypeStruct
    return pl.pallas_call(
        _tc_y_body,
        grid=(_GRID,),
        in_specs=[_row_spec(), _full_spec((H, H)), _full_spec((H,)),
                  _deg_spec(), _deg_spec()],
        out_specs=[_row_spec(), _row_spec()],
        out_shape=[st((N, H), jnp.float32), st((N, H), jnp.float32)],
    )(xin, w, b, deg0, deg1)


def _tc_hrelu(acca, accb, sb, deg0, deg1):
    st = jax.ShapeDtypeStruct
    return pl.pallas_call(
        _tc_hrelu_body,
        grid=(_GRID,),
        in_specs=[_row_spec(), _row_spec(), _row_spec(), _deg_spec(), _deg_spec()],
        out_specs=[_row_spec()],
        out_shape=[st((N, H), jnp.float32)],
    )(acca, accb, sb, deg0, deg1)[0]


def kernel(x, edge_index, batch, Wg0, bg0, Wg1, bg1, Wg2, bg2, Wg3, bg3,
           Wg4, bg4, Wh0, bh0, Wh1, bh1, Wh2, bh2):
    f32 = jnp.float32
    src3 = edge_index[0].reshape(NWORK, ROUNDS, CHUNK)
    dst3 = edge_index[1].reshape(NWORK, ROUNDS, CHUNK)
    src4 = edge_index[0].reshape(NWORK, EPW)
    dst4 = edge_index[1].reshape(NWORK, EPW)
    batch2 = batch.reshape(N, 1)

    lane0 = (jnp.arange(H) == 0).astype(f32)
    ones_hist = jnp.broadcast_to(lane0, (CHUNK, H))
    zeros_nh = jnp.zeros((NPAD, H), f32)
    zeros_c = jnp.zeros((NC80 * NC80, H), f32)

    degp = _sc_deg_kernel()(dst3, ones_hist, zeros_nh)
    deg0, deg1 = degp[0, :N], degp[1, :N]

    st = jax.ShapeDtypeStruct
    y0, sb0 = _tc_y(x, Wg0, bg0, deg0, deg1)
    acc0 = _sc_agg_kernel()(y0, src3, dst3, zeros_nh)
    h1 = _tc_hrelu(acc0[0, :N], acc0[1, :N], sb0, deg0, deg1)
    y1, sb1 = _tc_y(h1, Wg1, bg1, deg0, deg1)
    acc1 = _sc_agg_kernel()(y1, src3, dst3, zeros_nh)
    h2 = _tc_hrelu(acc1[0, :N], acc1[1, :N], sb1, deg0, deg1)
    seg2, cents = _tc_kmeans(h2, batch2)
    seg = seg2.reshape(N)
    hc = cents.transpose(1, 0, 2).reshape(NC80, H)

    cnt = _sc_count_kernel()(seg, src4, dst4, ones_hist, zeros_c)
    c0 = cnt[0, :, 0].reshape(NC80, NC80)
    c1 = cnt[1, :, 0].reshape(NC80, NC80)

    out = _tc_call(_tc_coarse_body, st((NB, 10), f32),
                   c0, c1, hc, Wg2, bg2, Wg3, bg3, Wg4, bg4,
                   Wh0, bh0, Wh1, bh1, Wh2, bh2)
    return out


# speedup vs baseline: 18.6950x; 1.1073x over previous
import functools

import jax
import jax.numpy as jnp
from jax import lax
from jax.experimental import pallas as pl
from jax.experimental.pallas import tpu as pltpu
from jax.experimental.pallas import tpu_sc as plsc

N = 10000
E = 320000
H = 128
NB = 16
NCL = 5
NC80 = NB * NCL
KM_ITERS = 10

NPAD = 10240
NWORK = 32
EPW = E // NWORK
CHUNK = 80
ROUNDS = EPW // CHUNK

_HIGH = jax.lax.Precision.HIGHEST


def _dot(a, b, dims):
    return jax.lax.dot_general(a, b, (dims, ((), ())), precision=_HIGH,
                               preferred_element_type=jnp.float32)


@functools.cache
def _sc_deg_kernel():
    mesh = plsc.VectorSubcoreMesh(core_axis_name="c", subcore_axis_name="s")
    return functools.partial(
        pl.kernel, mesh=mesh,
        out_type=jax.ShapeDtypeStruct((2, NPAD, H), jnp.float32),
        scratch_types=[
            pltpu.VMEM((ROUNDS, CHUNK), jnp.int32),
            pltpu.VMEM((CHUNK, H), jnp.float32),
            pltpu.VMEM_SHARED((NPAD, H), jnp.float32),
        ],
    )(_sc_deg_body)


def _sc_deg_body(dst3, ones_h, zeros_h, out, dstv, onesv, acc):
    c = lax.axis_index("c")
    s = lax.axis_index("s")
    w = c * 16 + s
    rows = NPAD // 16
    pltpu.sync_copy(dst3.at[w], dstv)
    pltpu.sync_copy(ones_h, onesv)
    pltpu.sync_copy(zeros_h.at[pl.ds(s * rows, rows)], acc.at[pl.ds(s * rows, rows)])
    plsc.subcore_barrier()

    def body(j, carry):
        pltpu.sync_copy(onesv, acc.at[dstv.at[j]], add=True)
        return carry

    lax.fori_loop(0, ROUNDS, body, 0)
    plsc.subcore_barrier()
    pltpu.sync_copy(acc.at[pl.ds(s * rows, rows)], out.at[c, pl.ds(s * rows, rows)])


@functools.cache
def _sc_agg_kernel():
    mesh = plsc.VectorSubcoreMesh(core_axis_name="c", subcore_axis_name="s")
    return functools.partial(
        pl.kernel, mesh=mesh,
        out_type=jax.ShapeDtypeStruct((2, NPAD, H), jnp.float32),
        scratch_types=[
            pltpu.VMEM((ROUNDS, CHUNK), jnp.int32),
            pltpu.VMEM((ROUNDS, CHUNK), jnp.int32),
            pltpu.VMEM((CHUNK, H), jnp.float32),
            pltpu.VMEM_SHARED((NPAD, H), jnp.float32),
            pltpu.SemaphoreType.DMA,
        ],
    )(_sc_agg_body)


def _sc_agg_body(y, src3, dst3, zeros_h, out, srcv, dstv, rowsv, acc, sem):
    c = lax.axis_index("c")
    s = lax.axis_index("s")
    w = c * 16 + s
    rows = NPAD // 16
    pltpu.sync_copy(src3.at[w], srcv)
    pltpu.sync_copy(dst3.at[w], dstv)
    pltpu.sync_copy(zeros_h.at[pl.ds(s * rows, rows)], acc.at[pl.ds(s * rows, rows)])
    plsc.subcore_barrier()

    def body(j, carry):
        pltpu.async_copy(y.at[srcv.at[j]], rowsv, sem).wait()
        pltpu.sync_copy(rowsv, acc.at[dstv.at[j]], add=True)
        return carry

    lax.fori_loop(0, ROUNDS, body, 0)
    plsc.subcore_barrier()
    pltpu.sync_copy(acc.at[pl.ds(s * rows, rows)], out.at[c, pl.ds(s * rows, rows)])


@functools.cache
def _sc_count_kernel():
    mesh = plsc.VectorSubcoreMesh(core_axis_name="c", subcore_axis_name="s")
    return functools.partial(
        pl.kernel, mesh=mesh,
        compiler_params=pltpu.CompilerParams(needs_layout_passes=False),
        out_type=jax.ShapeDtypeStruct((2, NC80 * NC80, H), jnp.float32),
        scratch_types=[
            pltpu.VMEM((N,), jnp.int32),
            pltpu.VMEM((EPW,), jnp.int32),
            pltpu.VMEM((EPW,), jnp.int32),
            pltpu.VMEM((1, CHUNK), jnp.int32),
            pltpu.VMEM((CHUNK, H), jnp.float32),
            pltpu.VMEM_SHARED((NC80 * NC80, H), jnp.float32),
        ],
    )(_sc_count_body)


def _sc_count_body(seg, src4, dst4, ones_h, zeros_h, out, segv, srcv, dstv, idxv,
                   onesv, acc):
    c = lax.axis_index("c")
    s = lax.axis_index("s")
    w = c * 16 + s
    rows = NC80 * NC80 // 16
    pltpu.sync_copy(seg, segv)
    pltpu.sync_copy(src4.at[w], srcv)
    pltpu.sync_copy(dst4.at[w], dstv)
    pltpu.sync_copy(ones_h, onesv)
    pltpu.sync_copy(zeros_h.at[pl.ds(s * rows, rows)], acc.at[pl.ds(s * rows, rows)])
    plsc.subcore_barrier()

    groups = CHUNK // 16

    def body(j, carry):
        for g in range(groups):
            base = (j * groups + g) * 16
            s16 = srcv[pl.ds(base, 16)]
            d16 = dstv[pl.ds(base, 16)]
            cs = plsc.load_gather(segv, [s16])
            cd = plsc.load_gather(segv, [d16])
            idxv.at[0][pl.ds(g * 16, 16)] = cs * NC80 + cd
        pltpu.sync_copy(onesv, acc.at[idxv.at[0]], add=True)
        return carry

    lax.fori_loop(0, ROUNDS, body, 0)
    plsc.subcore_barrier()
    pltpu.sync_copy(acc.at[pl.ds(s * rows, rows)], out.at[c, pl.ds(s * rows, rows)])


def _rs_from_deg(deg0, deg1):
    deg = deg0[:, 0:1] + deg1[:, 0:1] + 1.0
    rs = jax.lax.rsqrt(deg)
    return rs, rs * rs


def _tc_y_body(x, w0, b0, deg0, deg1, y_out, sb_out):
    rs, invd = _rs_from_deg(deg0[...], deg1[...])
    xw = _dot(x[...], w0[...], ((1,), (0,)))
    y_out[...] = xw * rs
    sb_out[...] = xw * invd + b0[...][None, :]


def _tc_hrelu_body(acca, accb, sb, deg0, deg1, h_out):
    rs, _ = _rs_from_deg(deg0[...], deg1[...])
    h_out[...] = jax.nn.relu(rs * (acca[...] + accb[...]) + sb[...])


_KB = 2000
_KGRID = N // _KB


def _tc_kmeans_body(h_ref, bat_ref, seg_out, cent_out, sums_scr, cnts_scr,
                    cent_scr):
    t = pl.program_id(0)
    b = pl.program_id(1)
    h = h_ref[...]
    bat = bat_ref[...]
    iota_g = jax.lax.broadcasted_iota(jnp.int32, (_KB, NB), 1)
    oh_g = jnp.where(bat == iota_g, 1.0, 0.0)
    iota80 = jax.lax.broadcasted_iota(jnp.int32, (_KB, NC80), 1)
    ones_col = jnp.ones((_KB, 1), jnp.float32)
    rowid = jax.lax.broadcasted_iota(jnp.int32, (_KB, 1), 0) + b * _KB
    init_lab = rowid % NCL

    cjall = _dot(oh_g, cent_scr[...], ((1,), (0,)))
    dmin = None
    lab = jnp.zeros((_KB, 1), jnp.int32)
    for j in range(NCL):
        diff = h - cjall[:, j * H:(j + 1) * H]
        dj = jnp.sum(diff * diff, axis=1, keepdims=True)
        if dmin is None:
            dmin = dj
        else:
            take = dj < dmin
            lab = jnp.where(take, j, lab)
            dmin = jnp.where(take, dj, dmin)
    label = jnp.where(t == 0, init_lab, lab)
    seg_out[...] = bat * NCL + label

    a80 = jnp.where(label * NB + bat == iota80, 1.0, 0.0)
    part = _dot(a80, h, ((0,), (0,)))
    pc = _dot(a80, ones_col, ((0,), (0,)))

    @pl.when(b == 0)
    def _():
        sums_scr[...] = part
        cnts_scr[...] = pc

    @pl.when(b > 0)
    def _():
        sums_scr[...] = sums_scr[...] + part
        cnts_scr[...] = cnts_scr[...] + pc

    @pl.when(b == _KGRID - 1)
    def _():
        cent80 = sums_scr[...] / jnp.maximum(cnts_scr[...], 1.0)
        for j in range(NCL):
            cent_scr[:, j * H:(j + 1) * H] = cent80[j * NB:(j + 1) * NB]

    @pl.when((t == KM_ITERS) & (b == _KGRID - 1))
    def _():
        cent80 = sums_scr[...] / jnp.maximum(cnts_scr[...], 1.0)
        for j in range(NCL):
            cent_out[j] = cent80[j * NB:(j + 1) * NB]


def _tc_kmeans(h2, batch2):
    st = jax.ShapeDtypeStruct
    return pl.pallas_call(
        _tc_kmeans_body,
        grid=(KM_ITERS + 1, _KGRID),
        in_specs=[pl.BlockSpec((_KB, H), lambda t, b: (b, 0)),
                  pl.BlockSpec((_KB, 1), lambda t, b: (b, 0))],
        out_specs=[pl.BlockSpec((_KB, 1), lambda t, b: (b, 0)),
                   pl.BlockSpec((NCL, NB, H), lambda t, b: (0, 0, 0))],
        out_shape=[st((N, 1), jnp.int32), st((NCL, NB, H), jnp.float32)],
        scratch_shapes=[pltpu.VMEM((NC80, H), jnp.float32),
                        pltpu.VMEM((NC80, 1), jnp.float32),
                        pltpu.VMEM((NB, NCL * H), jnp.float32)],
    )(h2, batch2)


def _tc_coarse_body(c0, c1, hc, w2, b2, w3, b3, w4, b4, wh0, bh0, wh1, bh1,
                    wh2, bh2, out):
    craw = c0[...] + c1[...]
    r = jax.lax.broadcasted_iota(jnp.int32, (NC80, NC80), 0)
    col = jax.lax.broadcasted_iota(jnp.int32, (NC80, NC80), 1)
    ct = jnp.where(r == col, 0.0, craw)
    ones_col = jnp.ones((NC80, 1), jnp.float32)
    degc = _dot(ct, ones_col, ((0,), (0,))) + 1.0
    degr = jnp.sum(ct, axis=0, keepdims=True) + 1.0
    rsc_c = jax.lax.rsqrt(degc)
    rsc_r = jax.lax.rsqrt(degr)
    invdc = rsc_c * rsc_c
    m = ct * rsc_c * rsc_r

    def layer(hm, w, b):
        xw = _dot(hm, w[...], ((1,), (0,)))
        agg = _dot(m, xw, ((0,), (0,)))
        return jax.nn.relu(agg + xw * invdc + b[...][None, :])

    h = layer(hc[...], w2, b2)
    h = layer(h, w3, b3)
    h = layer(h, w4, b4)

    pr = jax.lax.broadcasted_iota(jnp.int32, (NB, NC80), 0)
    pc = jax.lax.broadcasted_iota(jnp.int32, (NB, NC80), 1)
    pool = jnp.where(pc // NCL == pr, 1.0 / NCL, 0.0)
    g = _dot(pool, h, ((1,), (0,)))
    g = jax.nn.gelu(_dot(g, wh0[...], ((1,), (0,))) + bh0[...][None, :])
    g = jax.nn.gelu(_dot(g, wh1[...], ((1,), (0,))) + bh1[...][None, :])
    out[...] = _dot(g, wh2[...], ((1,), (0,))) + bh2[...][None, :]


def _tc_call(body, out_shapes, *args):
    return pl.pallas_call(body, out_shape=out_shapes)(*args)


_RB = 2000
_GRID = N // _RB


def _row_spec():
    return pl.BlockSpec((_RB, H), lambda i: (i, 0))


def _deg_spec():
    return pl.BlockSpec((_RB, H), lambda i: (i, 0))


def _full_spec(shape):
    return pl.BlockSpec(shape, lambda i: tuple(0 for _ in shape))


def _tc_y(xin, w, b, deg0, deg1):
    st = jax.ShapeDtypeStruct
    return pl.pallas_call(
        _tc_y_body,
        grid=(_GRID,),
        in_specs=[_row_spec(), _full_spec((H, H)), _full_spec((H,)),
                  _deg_spec(), _deg_spec()],
        out_specs=[_row_spec(), _row_spec()],
        out_shape=[st((N, H), jnp.float32), st((N, H), jnp.float32)],
    )(xin, w, b, deg0, deg1)


def _tc_hrelu(acca, accb, sb, deg0, deg1):
    st = jax.ShapeDtypeStruct
    return pl.pallas_call(
        _tc_hrelu_body,
        grid=(_GRID,),
        in_specs=[_row_spec(), _row_spec(), _row_spec(), _deg_spec(), _deg_spec()],
        out_specs=[_row_spec()],
        out_shape=[st((N, H), jnp.float32)],
    )(acca, accb, sb, deg0, deg1)[0]


def kernel(x, edge_index, batch, Wg0, bg0, Wg1, bg1, Wg2, bg2, Wg3, bg3,
           Wg4, bg4, Wh0, bh0, Wh1, bh1, Wh2, bh2):
    f32 = jnp.float32
    src3 = edge_index[0].reshape(NWORK, ROUNDS, CHUNK)
    dst3 = edge_index[1].reshape(NWORK, ROUNDS, CHUNK)
    src4 = edge_index[0].reshape(NWORK, EPW)
    dst4 = edge_index[1].reshape(NWORK, EPW)
    batch2 = batch.reshape(N, 1)

    lane0 = (jnp.arange(H) == 0).astype(f32)
    ones_hist = jnp.broadcast_to(lane0, (CHUNK, H))
    zeros_nh = jnp.zeros((NPAD, H), f32)
    zeros_c = jnp.zeros((NC80 * NC80, H), f32)

    degp = _sc_deg_kernel()(dst3, ones_hist, zeros_nh)
    deg0, deg1 = degp[0, :N], degp[1, :N]

    st = jax.ShapeDtypeStruct
    y0, sb0 = _tc_y(x, Wg0, bg0, deg0, deg1)
    acc0 = _sc_agg_kernel()(y0, src3, dst3, zeros_nh)
    h1 = _tc_hrelu(acc0[0, :N], acc0[1, :N], sb0, deg0, deg1)
    y1, sb1 = _tc_y(h1, Wg1, bg1, deg0, deg1)
    acc1 = _sc_agg_kernel()(y1, src3, dst3, zeros_nh)
    h2 = _tc_hrelu(acc1[0, :N], acc1[1, :N], sb1, deg0, deg1)
    seg2, cents = _tc_kmeans(h2, batch2)
    seg = seg2.reshape(N)
    hc = cents.transpose(1, 0, 2).reshape(NC80, H)

    cnt = _sc_count_kernel()(seg, src4, dst4, ones_hist, zeros_c)
    c0 = cnt[0, :, 0].reshape(NC80, NC80)
    c1 = cnt[1, :, 0].reshape(NC80, NC80)

    out = _tc_call(_tc_coarse_body, st((NB, 10), f32),
                   c0, c1, hc, Wg2, bg2, Wg3, bg3, Wg4, bg4,
                   Wh0, bh0, Wh1, bh1, Wh2, bh2)
    return out


# speedup vs baseline: 22.0534x; 1.1796x over previous
import functools

import jax
import jax.numpy as jnp
from jax import lax
from jax.experimental import pallas as pl
from jax.experimental.pallas import tpu as pltpu
from jax.experimental.pallas import tpu_sc as plsc

N = 10000
E = 320000
H = 128
NB = 16
NCL = 5
NC80 = NB * NCL
KM_ITERS = 10

NPAD = 10240
NWORK = 32
EPW = E // NWORK
CHUNK = 80
ROUNDS = EPW // CHUNK

_HIGH = jax.lax.Precision.HIGHEST


def _dot(a, b, dims):
    return jax.lax.dot_general(a, b, (dims, ((), ())), precision=_HIGH,
                               preferred_element_type=jnp.float32)


@functools.cache
def _sc_deg_kernel():
    mesh = plsc.VectorSubcoreMesh(core_axis_name="c", subcore_axis_name="s")
    return functools.partial(
        pl.kernel, mesh=mesh,
        out_type=jax.ShapeDtypeStruct((2, NPAD, H), jnp.float32),
        scratch_types=[
            pltpu.VMEM((ROUNDS, CHUNK), jnp.int32),
            pltpu.VMEM((CHUNK, H), jnp.float32),
            pltpu.VMEM_SHARED((NPAD, H), jnp.float32),
        ],
    )(_sc_deg_body)


def _sc_deg_body(dst3, ones_h, zeros_h, out, dstv, onesv, acc):
    c = lax.axis_index("c")
    s = lax.axis_index("s")
    w = c * 16 + s
    rows = NPAD // 16
    pltpu.sync_copy(dst3.at[w], dstv)
    pltpu.sync_copy(ones_h, onesv)
    pltpu.sync_copy(zeros_h.at[pl.ds(s * rows, rows)], acc.at[pl.ds(s * rows, rows)])
    plsc.subcore_barrier()

    def body(j, carry):
        pltpu.sync_copy(onesv, acc.at[dstv.at[j]], add=True)
        return carry

    lax.fori_loop(0, ROUNDS, body, 0)
    plsc.subcore_barrier()
    pltpu.sync_copy(acc.at[pl.ds(s * rows, rows)], out.at[c, pl.ds(s * rows, rows)])


@functools.cache
def _sc_agg_kernel():
    mesh = plsc.VectorSubcoreMesh(core_axis_name="c", subcore_axis_name="s")
    return functools.partial(
        pl.kernel, mesh=mesh,
        out_type=jax.ShapeDtypeStruct((2, NPAD, H), jnp.float32),
        scratch_types=[
            pltpu.VMEM((EPW,), jnp.int32),
            pltpu.VMEM((ROUNDS, CHUNK), jnp.int32),
            pltpu.VMEM((CHUNK, H), jnp.float32),
            pltpu.VMEM((CHUNK, H), jnp.float32),
            pltpu.VMEM_SHARED((NPAD, H), jnp.float32),
            pltpu.SemaphoreType.DMA,
        ],
    )(_sc_agg_body)


def _sc_agg_body(y, src4, dst3, zeros_h, out, srcv, dstv, rowsa, rowsb, acc, sem):
    c = lax.axis_index("c")
    s = lax.axis_index("s")
    w = c * 16 + s
    rows = NPAD // 16
    pltpu.sync_copy(src4.at[w], srcv)
    pltpu.sync_copy(dst3.at[w], dstv)
    pltpu.sync_copy(zeros_h.at[pl.ds(s * rows, rows)], acc.at[pl.ds(s * rows, rows)])
    plsc.subcore_barrier()

    def sidx(j):
        return srcv.at[pl.ds(j * CHUNK, CHUNK)]

    pltpu.async_copy(y.at[sidx(0)], rowsa, sem)

    def drain(j, buf):
        pltpu.make_async_copy(y.at[sidx(j)], buf, sem).wait()

    def pair(p, carry):
        j0 = 2 * p
        pltpu.async_copy(y.at[sidx(j0 + 1)], rowsb, sem)
        drain(j0, rowsa)
        pltpu.sync_copy(rowsa, acc.at[dstv.at[j0]], add=True)
        pltpu.async_copy(y.at[sidx(j0 + 2)], rowsa, sem)
        drain(j0 + 1, rowsb)
        pltpu.sync_copy(rowsb, acc.at[dstv.at[j0 + 1]], add=True)
        return carry

    lax.fori_loop(0, (ROUNDS - 1) // 2, pair, 0)
    drain(ROUNDS - 1, rowsa)
    pltpu.sync_copy(rowsa, acc.at[dstv.at[ROUNDS - 1]], add=True)
    plsc.subcore_barrier()
    pltpu.sync_copy(acc.at[pl.ds(s * rows, rows)], out.at[c, pl.ds(s * rows, rows)])


@functools.cache
def _sc_count_kernel():
    mesh = plsc.VectorSubcoreMesh(core_axis_name="c", subcore_axis_name="s")
    return functools.partial(
        pl.kernel, mesh=mesh,
        compiler_params=pltpu.CompilerParams(needs_layout_passes=False),
        out_type=jax.ShapeDtypeStruct((2, NC80 * NC80, H), jnp.float32),
        scratch_types=[
            pltpu.VMEM((N,), jnp.int32),
            pltpu.VMEM((EPW,), jnp.int32),
            pltpu.VMEM((EPW,), jnp.int32),
            pltpu.VMEM((1, CHUNK), jnp.int32),
            pltpu.VMEM((CHUNK, H), jnp.float32),
            pltpu.VMEM_SHARED((NC80 * NC80, H), jnp.float32),
        ],
    )(_sc_count_body)


def _sc_count_body(seg, src4, dst4, ones_h, zeros_h, out, segv, srcv, dstv, idxv,
                   onesv, acc):
    c = lax.axis_index("c")
    s = lax.axis_index("s")
    w = c * 16 + s
    rows = NC80 * NC80 // 16
    pltpu.sync_copy(seg, segv)
    pltpu.sync_copy(src4.at[w], srcv)
    pltpu.sync_copy(dst4.at[w], dstv)
    pltpu.sync_copy(ones_h, onesv)
    pltpu.sync_copy(zeros_h.at[pl.ds(s * rows, rows)], acc.at[pl.ds(s * rows, rows)])
    plsc.subcore_barrier()

    groups = CHUNK // 16

    def body(j, carry):
        for g in range(groups):
            base = (j * groups + g) * 16
            s16 = srcv[pl.ds(base, 16)]
            d16 = dstv[pl.ds(base, 16)]
            cs = plsc.load_gather(segv, [s16])
            cd = plsc.load_gather(segv, [d16])
            idxv.at[0][pl.ds(g * 16, 16)] = cs * NC80 + cd
        pltpu.sync_copy(onesv, acc.at[idxv.at[0]], add=True)
        return carry

    lax.fori_loop(0, ROUNDS, body, 0)
    plsc.subcore_barrier()
    pltpu.sync_copy(acc.at[pl.ds(s * rows, rows)], out.at[c, pl.ds(s * rows, rows)])


def _rs_from_deg(deg0, deg1):
    deg = deg0[:, 0:1] + deg1[:, 0:1] + 1.0
    rs = jax.lax.rsqrt(deg)
    return rs, rs * rs


def _tc_y_body(x, w0, b0, deg0, deg1, y_out, sb_out):
    rs, invd = _rs_from_deg(deg0[...], deg1[...])
    xw = _dot(x[...], w0[...], ((1,), (0,)))
    y_out[...] = xw * rs
    sb_out[...] = xw * invd + b0[...][None, :]


def _tc_hrelu_body(acca, accb, sb, deg0, deg1, h_out):
    rs, _ = _rs_from_deg(deg0[...], deg1[...])
    h_out[...] = jax.nn.relu(rs * (acca[...] + accb[...]) + sb[...])


_KB = 2000
_KGRID = N // _KB


def _tc_kmeans_body(h_ref, bat_ref, seg_out, cent_out, sums_scr, cnts_scr,
                    cent_scr):
    t = pl.program_id(0)
    b = pl.program_id(1)
    h = h_ref[...]
    bat = bat_ref[...]
    iota_g = jax.lax.broadcasted_iota(jnp.int32, (_KB, NB), 1)
    oh_g = jnp.where(bat == iota_g, 1.0, 0.0)
    iota80 = jax.lax.broadcasted_iota(jnp.int32, (_KB, NC80), 1)
    ones_col = jnp.ones((_KB, 1), jnp.float32)
    rowid = jax.lax.broadcasted_iota(jnp.int32, (_KB, 1), 0) + b * _KB
    init_lab = rowid % NCL

    cjall = _dot(oh_g, cent_scr[...], ((1,), (0,)))
    dmin = None
    lab = jnp.zeros((_KB, 1), jnp.int32)
    for j in range(NCL):
        diff = h - cjall[:, j * H:(j + 1) * H]
        dj = jnp.sum(diff * diff, axis=1, keepdims=True)
        if dmin is None:
            dmin = dj
        else:
            take = dj < dmin
            lab = jnp.where(take, j, lab)
            dmin = jnp.where(take, dj, dmin)
    label = jnp.where(t == 0, init_lab, lab)
    seg_out[...] = bat * NCL + label

    a80 = jnp.where(label * NB + bat == iota80, 1.0, 0.0)
    part = _dot(a80, h, ((0,), (0,)))
    pc = _dot(a80, ones_col, ((0,), (0,)))

    @pl.when(b == 0)
    def _():
        sums_scr[...] = part
        cnts_scr[...] = pc

    @pl.when(b > 0)
    def _():
        sums_scr[...] = sums_scr[...] + part
        cnts_scr[...] = cnts_scr[...] + pc

    @pl.when(b == _KGRID - 1)
    def _():
        cent80 = sums_scr[...] / jnp.maximum(cnts_scr[...], 1.0)
        for j in range(NCL):
            cent_scr[:, j * H:(j + 1) * H] = cent80[j * NB:(j + 1) * NB]

    @pl.when((t == KM_ITERS) & (b == _KGRID - 1))
    def _():
        cent80 = sums_scr[...] / jnp.maximum(cnts_scr[...], 1.0)
        for j in range(NCL):
            cent_out[j] = cent80[j * NB:(j + 1) * NB]


def _tc_kmeans(h2, batch2):
    st = jax.ShapeDtypeStruct
    return pl.pallas_call(
        _tc_kmeans_body,
        grid=(KM_ITERS + 1, _KGRID),
        in_specs=[pl.BlockSpec((_KB, H), lambda t, b: (b, 0)),
                  pl.BlockSpec((_KB, 1), lambda t, b: (b, 0))],
        out_specs=[pl.BlockSpec((_KB, 1), lambda t, b: (b, 0)),
                   pl.BlockSpec((NCL, NB, H), lambda t, b: (0, 0, 0))],
        out_shape=[st((N, 1), jnp.int32), st((NCL, NB, H), jnp.float32)],
        scratch_shapes=[pltpu.VMEM((NC80, H), jnp.float32),
                        pltpu.VMEM((NC80, 1), jnp.float32),
                        pltpu.VMEM((NB, NCL * H), jnp.float32)],
    )(h2, batch2)


def _tc_coarse_body(c0, c1, hc, w2, b2, w3, b3, w4, b4, wh0, bh0, wh1, bh1,
                    wh2, bh2, out):
    craw = c0[...] + c1[...]
    r = jax.lax.broadcasted_iota(jnp.int32, (NC80, NC80), 0)
    col = jax.lax.broadcasted_iota(jnp.int32, (NC80, NC80), 1)
    ct = jnp.where(r == col, 0.0, craw)
    ones_col = jnp.ones((NC80, 1), jnp.float32)
    degc = _dot(ct, ones_col, ((0,), (0,))) + 1.0
    degr = jnp.sum(ct, axis=0, keepdims=True) + 1.0
    rsc_c = jax.lax.rsqrt(degc)
    rsc_r = jax.lax.rsqrt(degr)
    invdc = rsc_c * rsc_c
    m = ct * rsc_c * rsc_r

    def layer(hm, w, b):
        xw = _dot(hm, w[...], ((1,), (0,)))
        agg = _dot(m, xw, ((0,), (0,)))
        return jax.nn.relu(agg + xw * invdc + b[...][None, :])

    h = layer(hc[...], w2, b2)
    h = layer(h, w3, b3)
    h = layer(h, w4, b4)

    pr = jax.lax.broadcasted_iota(jnp.int32, (NB, NC80), 0)
    pc = jax.lax.broadcasted_iota(jnp.int32, (NB, NC80), 1)
    pool = jnp.where(pc // NCL == pr, 1.0 / NCL, 0.0)
    g = _dot(pool, h, ((1,), (0,)))
    g = jax.nn.gelu(_dot(g, wh0[...], ((1,), (0,))) + bh0[...][None, :])
    g = jax.nn.gelu(_dot(g, wh1[...], ((1,), (0,))) + bh1[...][None, :])
    out[...] = _dot(g, wh2[...], ((1,), (0,))) + bh2[...][None, :]


def _tc_call(body, out_shapes, *args):
    return pl.pallas_call(body, out_shape=out_shapes)(*args)


_RB = 2000
_GRID = N // _RB


def _row_spec():
    return pl.BlockSpec((_RB, H), lambda i: (i, 0))


def _deg_spec():
    return pl.BlockSpec((_RB, H), lambda i: (i, 0))


def _full_spec(shape):
    return pl.BlockSpec(shape, lambda i: tuple(0 for _ in shape))


def _tc_y(xin, w, b, deg0, deg1):
    st = jax.ShapeDtypeStruct
    return pl.pallas_call(
        _tc_y_body,
        grid=(_GRID,),
        in_specs=[_row_spec(), _full_spec((H, H)), _full_spec((H,)),
                  _deg_spec(), _deg_spec()],
        out_specs=[_row_spec(), _row_spec()],
        out_shape=[st((N, H), jnp.float32), st((N, H), jnp.float32)],
    )(xin, w, b, deg0, deg1)


def _tc_hrelu(acca, accb, sb, deg0, deg1):
    st = jax.ShapeDtypeStruct
    return pl.pallas_call(
        _tc_hrelu_body,
        grid=(_GRID,),
        in_specs=[_row_spec(), _row_spec(), _row_spec(), _deg_spec(), _deg_spec()],
        out_specs=[_row_spec()],
        out_shape=[st((N, H), jnp.float32)],
    )(acca, accb, sb, deg0, deg1)[0]


def kernel(x, edge_index, batch, Wg0, bg0, Wg1, bg1, Wg2, bg2, Wg3, bg3,
           Wg4, bg4, Wh0, bh0, Wh1, bh1, Wh2, bh2):
    f32 = jnp.float32
    src3 = edge_index[0].reshape(NWORK, ROUNDS, CHUNK)
    dst3 = edge_index[1].reshape(NWORK, ROUNDS, CHUNK)
    src4 = edge_index[0].reshape(NWORK, EPW)
    dst4 = edge_index[1].reshape(NWORK, EPW)
    batch2 = batch.reshape(N, 1)

    lane0 = (jnp.arange(H) == 0).astype(f32)
    ones_hist = jnp.broadcast_to(lane0, (CHUNK, H))
    zeros_nh = jnp.zeros((NPAD, H), f32)
    zeros_c = jnp.zeros((NC80 * NC80, H), f32)

    degp = _sc_deg_kernel()(dst3, ones_hist, zeros_nh)
    deg0, deg1 = degp[0, :N], degp[1, :N]

    st = jax.ShapeDtypeStruct
    y0, sb0 = _tc_y(x, Wg0, bg0, deg0, deg1)
    acc0 = _sc_agg_kernel()(y0, src4, dst3, zeros_nh)
    h1 = _tc_hrelu(acc0[0, :N], acc0[1, :N], sb0, deg0, deg1)
    y1, sb1 = _tc_y(h1, Wg1, bg1, deg0, deg1)
    acc1 = _sc_agg_kernel()(y1, src4, dst3, zeros_nh)
    h2 = _tc_hrelu(acc1[0, :N], acc1[1, :N], sb1, deg0, deg1)
    seg2, cents = _tc_kmeans(h2, batch2)
    seg = seg2.reshape(N)
    hc = cents.transpose(1, 0, 2).reshape(NC80, H)

    cnt = _sc_count_kernel()(seg, src4, dst4, ones_hist, zeros_c)
    c0 = cnt[0, :, 0].reshape(NC80, NC80)
    c1 = cnt[1, :, 0].reshape(NC80, NC80)

    out = _tc_call(_tc_coarse_body, st((NB, 10), f32),
                   c0, c1, hc, Wg2, bg2, Wg3, bg3, Wg4, bg4,
                   Wh0, bh0, Wh1, bh1, Wh2, bh2)
    return out
